# baseline (speedup 1.0000x reference)
import functools

import jax
import jax.numpy as jnp
from jax import lax
from jax.experimental import pallas as pl
from jax.experimental.pallas import tpu as pltpu
from jax.experimental.pallas import tpu_sc as plsc

N_NODES = 10000
NPAD = 10240
D = 256
DH = 128
E = 160000
K = 125
ROWS = E // K
NS = 16
TROWS = ROWS // NS
EPT = E // NS
NPT = N_NODES // NS
G = 16
CG = 2000
CROWS = NPAD // DH


def _sc_segment_sum(x, src2, dst2, dst1):
    mesh = plsc.VectorSubcoreMesh(core_axis_name="c", subcore_axis_name="s")

    @functools.partial(
        pl.kernel,
        mesh=mesh,
        compiler_params=pltpu.CompilerParams(use_tc_tiling_on_sc=False,
                                             needs_layout_passes=False),
        out_type=(
            jax.ShapeDtypeStruct((N_NODES, DH), jnp.float32),
            jax.ShapeDtypeStruct((N_NODES, DH), jnp.float32),
            jax.ShapeDtypeStruct((CROWS, DH), jnp.float32),
            jax.ShapeDtypeStruct((N_NODES, DH), jnp.float32),
            jax.ShapeDtypeStruct((N_NODES, DH), jnp.float32),
        ),
        scratch_types=[
            pltpu.VMEM((G, K), jnp.int32),
            pltpu.VMEM((G, K), jnp.int32),
            pltpu.VMEM((K, DH), jnp.float32),
            pltpu.VMEM((K, DH), jnp.float32),
            pltpu.VMEM((CG,), jnp.int32),
            pltpu.VMEM((CROWS, DH), jnp.float32),
            pltpu.VMEM((CROWS,), jnp.int32),
            pltpu.VMEM_SHARED((N_NODES, DH), jnp.float32),
            pltpu.VMEM_SHARED((CROWS, DH), jnp.float32),
            pltpu.SemaphoreType.DMA,
            pltpu.SemaphoreType.DMA,
        ],
    )
    def k(x_hbm, src_hbm, dst_hbm, dst1_hbm, out0, out1, cnt_out, xh0, xh1,
          sidx, didx, rows, rows2, dchunk, blk, riota, acc, cnt_sp, sem, sem2):
        c = lax.axis_index("c")
        s = lax.axis_index("s")

        zero16 = jnp.zeros((16,), jnp.float32)
        one16 = jnp.ones((16,), jnp.float32)

        def zrow(i, carry):
            for j in range(DH // 16):
                rows[i, pl.ds(j * 16, 16)] = zero16
            return carry

        lax.fori_loop(0, K, zrow, 0)

        def brow(i, carry):
            for j in range(DH // 16):
                blk[i, pl.ds(j * 16, 16)] = zero16
            return carry

        lax.fori_loop(0, CROWS, brow, 0)

        for j in range(CROWS // 16):
            riota[pl.ds(j * 16, 16)] = lax.iota(jnp.int32, 16) + j * 16

        for j in range(NPT // K):
            pltpu.sync_copy(rows, acc.at[pl.ds(s * NPT + j * K, K)])

        @pl.when(s == 0)
        def _():
            pltpu.sync_copy(rows.at[pl.ds(0, CROWS)], cnt_sp)

        def stage_x(col0, xh_ref):
            for j in range(NPT // K):
                sl = pl.ds(s * NPT + j * K, K)
                pltpu.sync_copy(x_hbm.at[sl, pl.ds(col0, DH)], rows2)
                pltpu.sync_copy(rows2, xh_ref.at[sl])

        @pl.when(c == 0)
        def _():
            stage_x(0, xh0)

        @pl.when(c == 1)
        def _():
            stage_x(DH, xh1)

        plsc.subcore_barrier()

        def edge_loop(x_ref):
            def group(g, carry):
                pltpu.sync_copy(src_hbm.at[pl.ds(s * TROWS + g * G, G)], sidx)
                pltpu.sync_copy(dst_hbm.at[pl.ds(s * TROWS + g * G, G)], didx)
                pltpu.sync_copy(dst1_hbm.at[pl.ds(s * EPT + g * CG, CG)],
                                dchunk)
                pltpu.async_copy(x_ref.at[sidx.at[0]], rows, sem)

                def cbody(j, carry2):
                    idx = dchunk[pl.ds(j * 16, 16)]
                    plsc.addupdate_scatter(
                        blk,
                        [lax.shift_right_logical(idx, 7),
                         lax.bitwise_and(idx, 127)],
                        one16,
                    )
                    return carry2

                lax.fori_loop(0, CG // 16, cbody, 0)

                def pair(p, carry2):
                    pltpu.async_copy(x_ref.at[sidx.at[2 * p + 1]], rows2, sem2)
                    pltpu.make_async_copy(x_ref.at[sidx.at[2 * p]], rows,
                                          sem).wait()
                    pltpu.sync_copy(rows, acc.at[didx.at[2 * p]], add=True)

                    @pl.when(p < G // 2 - 1)
                    def _():
                        pltpu.async_copy(x_ref.at[sidx.at[2 * p + 2]], rows,
                                         sem)

                    pltpu.make_async_copy(x_ref.at[sidx.at[2 * p + 1]], rows2,
                                          sem2).wait()
                    pltpu.sync_copy(rows2, acc.at[didx.at[2 * p + 1]],
                                    add=True)
                    return carry2

                lax.fori_loop(0, G // 2, pair, 0)
                return carry

            lax.fori_loop(0, TROWS // G, group, 0)

        @pl.when(c == 0)
        def _():
            edge_loop(xh0)

        @pl.when(c == 1)
        def _():
            edge_loop(xh1)

        plsc.subcore_barrier()

        pltpu.sync_copy(blk, cnt_sp.at[riota], add=True)
        plsc.subcore_barrier()

        def copy_out(dst_hbm_ref):
            for j in range(NPT // K):
                pltpu.sync_copy(acc.at[pl.ds(s * NPT + j * K, K)], rows)
                pltpu.sync_copy(rows, dst_hbm_ref.at[pl.ds(s * NPT + j * K, K)])

        @pl.when(c == 0)
        def _():
            copy_out(out0)

        @pl.when(c == 1)
        def _():
            copy_out(out1)

        @pl.when(jnp.logical_and(c == 0, s == 0))
        def _():
            pltpu.sync_copy(cnt_sp, blk)
            pltpu.sync_copy(blk, cnt_out)

    return k(x, src2, dst2, dst1)[:3]


BM = 400


def _tc_body(x_ref, nb0_ref, nb1_ref, cnt_ref, wst_ref, wnt0_ref, wnt1_ref,
             b_ref, g_ref, be_ref, o_ref):
    hs = jnp.dot(x_ref[...], wst_ref[...], preferred_element_type=jnp.float32)
    hn = (jnp.dot(nb0_ref[...], wnt0_ref[...], preferred_element_type=jnp.float32)
          + jnp.dot(nb1_ref[...], wnt1_ref[...], preferred_element_type=jnp.float32))
    inv = 1.0 / jnp.maximum(cnt_ref[...], 1.0)
    h = hs + hn * inv + b_ref[...]
    mu = jnp.mean(h, axis=-1, keepdims=True)
    d = h - mu
    var = jnp.mean(d * d, axis=-1, keepdims=True)
    o_ref[...] = d * lax.rsqrt(var + 1e-5) * g_ref[...] + be_ref[...]


def _tc_combine(x, nb0, nb1, cnt, wst, wnt0, wnt1, bias, gamma, beta):
    grid = (N_NODES // BM,)
    return pl.pallas_call(
        _tc_body,
        grid=grid,
        in_specs=[
            pl.BlockSpec((BM, D), lambda i: (i, 0)),
            pl.BlockSpec((BM, DH), lambda i: (i, 0)),
            pl.BlockSpec((BM, DH), lambda i: (i, 0)),
            pl.BlockSpec((BM, 1), lambda i: (i, 0)),
            pl.BlockSpec((D, D), lambda i: (0, 0)),
            pl.BlockSpec((DH, D), lambda i: (0, 0)),
            pl.BlockSpec((DH, D), lambda i: (0, 0)),
            pl.BlockSpec((1, D), lambda i: (0, 0)),
            pl.BlockSpec((1, D), lambda i: (0, 0)),
            pl.BlockSpec((1, D), lambda i: (0, 0)),
        ],
        out_specs=pl.BlockSpec((BM, D), lambda i: (i, 0)),
        out_shape=jax.ShapeDtypeStruct((N_NODES, D), jnp.float32),
    )(x, nb0, nb1, cnt, wst, wnt0, wnt1, bias, gamma, beta)


@jax.jit
def kernel(x, edge_index, deg, W_self, W_neigh, bias, ln_gamma, ln_beta):
    del deg
    src2 = edge_index[1].reshape(ROWS, K)
    dst2 = edge_index[0].reshape(ROWS, K)
    dst1 = edge_index[0]
    nb0, nb1, cnt_tab = _sc_segment_sum(x, src2, dst2, dst1)
    cnt = cnt_tab.reshape(NPAD)[:N_NODES, None]
    wnt = W_neigh.T
    return _tc_combine(x, nb0, nb1, cnt, W_self.T, wnt[:DH], wnt[DH:],
                       bias[None, :], ln_gamma[None, :], ln_beta[None, :])


# speedup vs baseline: 1.0448x; 1.0448x over previous
import functools

import jax
import jax.numpy as jnp
from jax import lax
from jax.experimental import pallas as pl
from jax.experimental.pallas import tpu as pltpu
from jax.experimental.pallas import tpu_sc as plsc

N_NODES = 10000
NPAD = 10240
D = 256
DH = 128
E = 160000
K = 125
ROWS = E // K
NS = 16
TROWS = ROWS // NS
EPT = E // NS
NPT = N_NODES // NS
G = 16
CG = 2000
CROWS = NPAD // DH


def _sc_segment_sum(x0, x1, src2, dst2, dst1):
    mesh = plsc.VectorSubcoreMesh(core_axis_name="c", subcore_axis_name="s")

    @functools.partial(
        pl.kernel,
        mesh=mesh,
        compiler_params=pltpu.CompilerParams(use_tc_tiling_on_sc=False,
                                             needs_layout_passes=False),
        out_type=(
            jax.ShapeDtypeStruct((N_NODES, DH), jnp.float32),
            jax.ShapeDtypeStruct((N_NODES, DH), jnp.float32),
            jax.ShapeDtypeStruct((CROWS, DH), jnp.float32),
        ),
        scratch_types=[
            pltpu.VMEM((G, K), jnp.int32),
            pltpu.VMEM((G, K), jnp.int32),
            pltpu.VMEM((K, DH), jnp.float32),
            pltpu.VMEM((K, DH), jnp.float32),
            pltpu.VMEM((CG,), jnp.int32),
            pltpu.VMEM((CROWS, DH), jnp.float32),
            pltpu.VMEM((CROWS,), jnp.int32),
            pltpu.VMEM_SHARED((N_NODES, DH), jnp.float32),
            pltpu.VMEM_SHARED((CROWS, DH), jnp.float32),
            pltpu.SemaphoreType.DMA,
            pltpu.SemaphoreType.DMA,
        ],
    )
    def k(x0_hbm, x1_hbm, src_hbm, dst_hbm, dst1_hbm, out0, out1, cnt_out,
          sidx, didx, rows, rows2, dchunk, blk, riota, acc, cnt_sp, sem, sem2):
        c = lax.axis_index("c")
        s = lax.axis_index("s")

        zero16 = jnp.zeros((16,), jnp.float32)
        one16 = jnp.ones((16,), jnp.float32)

        def zrow(i, carry):
            for j in range(DH // 16):
                rows[i, pl.ds(j * 16, 16)] = zero16
            return carry

        lax.fori_loop(0, K, zrow, 0)

        def brow(i, carry):
            for j in range(DH // 16):
                blk[i, pl.ds(j * 16, 16)] = zero16
            return carry

        lax.fori_loop(0, CROWS, brow, 0)

        for j in range(CROWS // 16):
            riota[pl.ds(j * 16, 16)] = lax.iota(jnp.int32, 16) + j * 16

        for j in range(NPT // K):
            pltpu.sync_copy(rows, acc.at[pl.ds(s * NPT + j * K, K)])

        @pl.when(s == 0)
        def _():
            pltpu.sync_copy(rows.at[pl.ds(0, CROWS)], cnt_sp)

        plsc.subcore_barrier()

        def edge_loop(x_ref):
            def group(g, carry):
                pltpu.sync_copy(src_hbm.at[pl.ds(s * TROWS + g * G, G)], sidx)
                pltpu.sync_copy(dst_hbm.at[pl.ds(s * TROWS + g * G, G)], didx)
                pltpu.sync_copy(dst1_hbm.at[pl.ds(s * EPT + g * CG, CG)],
                                dchunk)
                pltpu.async_copy(x_ref.at[sidx.at[0]], rows, sem)

                def cbody(j, carry2):
                    idx = dchunk[pl.ds(j * 16, 16)]
                    plsc.addupdate_scatter(
                        blk,
                        [lax.shift_right_logical(idx, 7),
                         lax.bitwise_and(idx, 127)],
                        one16,
                    )
                    return carry2

                lax.fori_loop(0, CG // 16, cbody, 0)

                def pair(p, carry2):
                    pltpu.async_copy(x_ref.at[sidx.at[2 * p + 1]], rows2, sem2)
                    pltpu.make_async_copy(x_ref.at[sidx.at[2 * p]], rows,
                                          sem).wait()
                    pltpu.sync_copy(rows, acc.at[didx.at[2 * p]], add=True)

                    @pl.when(p < G // 2 - 1)
                    def _():
                        pltpu.async_copy(x_ref.at[sidx.at[2 * p + 2]], rows,
                                         sem)

                    pltpu.make_async_copy(x_ref.at[sidx.at[2 * p + 1]], rows2,
                                          sem2).wait()
                    pltpu.sync_copy(rows2, acc.at[didx.at[2 * p + 1]],
                                    add=True)
                    return carry2

                lax.fori_loop(0, G // 2, pair, 0)
                return carry

            lax.fori_loop(0, TROWS // G, group, 0)

        @pl.when(c == 0)
        def _():
            edge_loop(x0_hbm)

        @pl.when(c == 1)
        def _():
            edge_loop(x1_hbm)

        plsc.subcore_barrier()

        pltpu.sync_copy(blk, cnt_sp.at[riota], add=True)
        plsc.subcore_barrier()

        def copy_out(dst_hbm_ref):
            for j in range(NPT // K):
                pltpu.sync_copy(acc.at[pl.ds(s * NPT + j * K, K)], rows)
                pltpu.sync_copy(rows, dst_hbm_ref.at[pl.ds(s * NPT + j * K, K)])

        @pl.when(c == 0)
        def _():
            copy_out(out0)

        @pl.when(c == 1)
        def _():
            copy_out(out1)

        @pl.when(jnp.logical_and(c == 0, s == 0))
        def _():
            pltpu.sync_copy(cnt_sp, blk)
            pltpu.sync_copy(blk, cnt_out)

    return k(x0, x1, src2, dst2, dst1)


BM = 400


def _tc_body(x_ref, nb0_ref, nb1_ref, cnt_ref, wst_ref, wnt0_ref, wnt1_ref,
             b_ref, g_ref, be_ref, o_ref):
    hs = jnp.dot(x_ref[...], wst_ref[...], preferred_element_type=jnp.float32)
    hn = (jnp.dot(nb0_ref[...], wnt0_ref[...], preferred_element_type=jnp.float32)
          + jnp.dot(nb1_ref[...], wnt1_ref[...], preferred_element_type=jnp.float32))
    inv = 1.0 / jnp.maximum(cnt_ref[...], 1.0)
    h = hs + hn * inv + b_ref[...]
    mu = jnp.mean(h, axis=-1, keepdims=True)
    d = h - mu
    var = jnp.mean(d * d, axis=-1, keepdims=True)
    o_ref[...] = d * lax.rsqrt(var + 1e-5) * g_ref[...] + be_ref[...]


def _tc_combine(x, nb0, nb1, cnt, wst, wnt0, wnt1, bias, gamma, beta):
    grid = (N_NODES // BM,)
    return pl.pallas_call(
        _tc_body,
        grid=grid,
        in_specs=[
            pl.BlockSpec((BM, D), lambda i: (i, 0)),
            pl.BlockSpec((BM, DH), lambda i: (i, 0)),
            pl.BlockSpec((BM, DH), lambda i: (i, 0)),
            pl.BlockSpec((BM, 1), lambda i: (i, 0)),
            pl.BlockSpec((D, D), lambda i: (0, 0)),
            pl.BlockSpec((DH, D), lambda i: (0, 0)),
            pl.BlockSpec((DH, D), lambda i: (0, 0)),
            pl.BlockSpec((1, D), lambda i: (0, 0)),
            pl.BlockSpec((1, D), lambda i: (0, 0)),
            pl.BlockSpec((1, D), lambda i: (0, 0)),
        ],
        out_specs=pl.BlockSpec((BM, D), lambda i: (i, 0)),
        out_shape=jax.ShapeDtypeStruct((N_NODES, D), jnp.float32),
    )(x, nb0, nb1, cnt, wst, wnt0, wnt1, bias, gamma, beta)


@jax.jit
def kernel(x, edge_index, deg, W_self, W_neigh, bias, ln_gamma, ln_beta):
    del deg
    x0 = x[:, :DH]
    x1 = x[:, DH:]
    src2 = edge_index[1].reshape(ROWS, K)
    dst2 = edge_index[0].reshape(ROWS, K)
    dst1 = edge_index[0]
    nb0, nb1, cnt_tab = _sc_segment_sum(x0, x1, src2, dst2, dst1)
    cnt = cnt_tab.reshape(NPAD)[:N_NODES, None]
    wnt = W_neigh.T
    return _tc_combine(x, nb0, nb1, cnt, W_self.T, wnt[:DH], wnt[DH:],
                       bias[None, :], ln_gamma[None, :], ln_beta[None, :])


# speedup vs baseline: 1.0577x; 1.0123x over previous
import functools

import jax
import jax.numpy as jnp
from jax import lax
from jax.experimental import pallas as pl
from jax.experimental.pallas import tpu as pltpu
from jax.experimental.pallas import tpu_sc as plsc

N_NODES = 10000
NPAD = 10240
D = 256
DH = 128
E = 160000
K = 100
ROWS = E // K
NS = 16
TROWS = ROWS // NS
EPT = E // NS
NPT = N_NODES // NS
G = 20
NGRP = TROWS // G
CG = G * K
CROWS = NPAD // DH
OFFS = ((0, 100), (100, 100), (200, 100), (300, 100), (400, 100),
        (500, 100), (600, 25))


def _sc_segment_sum(x0, x1, src2, dst2, dst1):
    mesh = plsc.VectorSubcoreMesh(core_axis_name="c", subcore_axis_name="s")

    @functools.partial(
        pl.kernel,
        mesh=mesh,
        compiler_params=pltpu.CompilerParams(use_tc_tiling_on_sc=False,
                                             needs_layout_passes=False),
        out_type=(
            jax.ShapeDtypeStruct((N_NODES, DH), jnp.float32),
            jax.ShapeDtypeStruct((N_NODES, DH), jnp.float32),
            jax.ShapeDtypeStruct((CROWS, DH), jnp.float32),
        ),
        scratch_types=[
            pltpu.VMEM((G, K), jnp.int32),
            pltpu.VMEM((G, K), jnp.int32),
            pltpu.VMEM((G, K), jnp.int32),
            pltpu.VMEM((G, K), jnp.int32),
            pltpu.VMEM((CG,), jnp.int32),
            pltpu.VMEM((CG,), jnp.int32),
            pltpu.VMEM((K, DH), jnp.float32),
            pltpu.VMEM((K, DH), jnp.float32),
            pltpu.VMEM((CROWS, DH), jnp.float32),
            pltpu.VMEM((CROWS,), jnp.int32),
            pltpu.VMEM_SHARED((N_NODES, DH), jnp.float32),
            pltpu.VMEM_SHARED((CROWS, DH), jnp.float32),
            pltpu.SemaphoreType.DMA,
            pltpu.SemaphoreType.DMA,
            pltpu.SemaphoreType.DMA,
            pltpu.SemaphoreType.DMA,
        ],
    )
    def k(x0_hbm, x1_hbm, src_hbm, dst_hbm, dst1_hbm, out0, out1, cnt_out,
          sidxa, sidxb, didxa, didxb, dcha, dchb, rows, rows2, blk, riota,
          acc, cnt_sp, sem, sem2, semi, semw):
        c = lax.axis_index("c")
        s = lax.axis_index("s")

        zero16 = jnp.zeros((16,), jnp.float32)
        one16 = jnp.ones((16,), jnp.float32)

        def zrow(i, carry):
            for j in range(DH // 16):
                rows[i, pl.ds(j * 16, 16)] = zero16
            return carry

        lax.fori_loop(0, K, zrow, 0)

        def brow(i, carry):
            for j in range(DH // 16):
                blk[i, pl.ds(j * 16, 16)] = zero16
            return carry

        lax.fori_loop(0, CROWS, brow, 0)

        for j in range(CROWS // 16):
            riota[pl.ds(j * 16, 16)] = lax.iota(jnp.int32, 16) + j * 16

        for o, l in OFFS:
            pltpu.async_copy(rows.at[pl.ds(0, l)],
                             acc.at[pl.ds(s * NPT + o, l)], semw)

        @pl.when(s == 0)
        def _():
            pltpu.async_copy(rows.at[pl.ds(0, CROWS)], cnt_sp, semw)

        def load_idx(g, sb, db, cb):
            pltpu.sync_copy(src_hbm.at[pl.ds(s * TROWS + g * G, G)], sb)
            pltpu.sync_copy(dst_hbm.at[pl.ds(s * TROWS + g * G, G)], db)
            pltpu.sync_copy(dst1_hbm.at[pl.ds(s * EPT + g * CG, CG)], cb)

        def prefetch_idx(g, sb, db, cb):
            pltpu.async_copy(src_hbm.at[pl.ds(s * TROWS + g * G, G)], sb, semi)
            pltpu.async_copy(dst_hbm.at[pl.ds(s * TROWS + g * G, G)], db, semi)
            pltpu.async_copy(dst1_hbm.at[pl.ds(s * EPT + g * CG, CG)], cb,
                             semi)

        def wait_idx(g, sb, db, cb):
            pltpu.make_async_copy(src_hbm.at[pl.ds(s * TROWS + g * G, G)], sb,
                                  semi).wait()
            pltpu.make_async_copy(dst_hbm.at[pl.ds(s * TROWS + g * G, G)], db,
                                  semi).wait()
            pltpu.make_async_copy(dst1_hbm.at[pl.ds(s * EPT + g * CG, CG)],
                                  cb, semi).wait()

        load_idx(0, sidxa, didxa, dcha)

        for o, l in OFFS:
            pltpu.make_async_copy(rows.at[pl.ds(0, l)],
                                  acc.at[pl.ds(s * NPT + o, l)], semw).wait()

        @pl.when(s == 0)
        def _():
            pltpu.make_async_copy(rows.at[pl.ds(0, CROWS)], cnt_sp,
                                  semw).wait()

        plsc.subcore_barrier()

        def edge_loop(x_ref):
            pltpu.async_copy(x_ref.at[sidxa.at[0]], rows, sem)
            bufs = (sidxa, didxa, dcha), (sidxb, didxb, dchb)
            for g in range(NGRP):
                cs, cd, cc = bufs[g % 2]
                ns, nd, nc = bufs[(g + 1) % 2]
                if g + 1 < NGRP:
                    prefetch_idx(g + 1, ns, nd, nc)

                def cbody(j, carry2):
                    idx = cc[pl.ds(j * 16, 16)]
                    plsc.addupdate_scatter(
                        blk,
                        [lax.shift_right_logical(idx, 7),
                         lax.bitwise_and(idx, 127)],
                        one16,
                    )
                    return carry2

                lax.fori_loop(0, CG // 16, cbody, 0)

                def pair(p, carry2):
                    pltpu.async_copy(x_ref.at[cs.at[2 * p + 1]], rows2, sem2)
                    pltpu.make_async_copy(x_ref.at[cs.at[2 * p]], rows,
                                          sem).wait()
                    pltpu.sync_copy(rows, acc.at[cd.at[2 * p]], add=True)

                    @pl.when(p < G // 2 - 1)
                    def _():
                        pltpu.async_copy(x_ref.at[cs.at[2 * p + 2]], rows,
                                         sem)

                    pltpu.make_async_copy(x_ref.at[cs.at[2 * p + 1]], rows2,
                                          sem2).wait()
                    pltpu.sync_copy(rows2, acc.at[cd.at[2 * p + 1]],
                                    add=True)
                    return carry2

                lax.fori_loop(0, G // 2, pair, 0)

                if g + 1 < NGRP:
                    wait_idx(g + 1, ns, nd, nc)
                    pltpu.async_copy(x_ref.at[ns.at[0]], rows, sem)

        @pl.when(c == 0)
        def _():
            edge_loop(x0_hbm)

        @pl.when(c == 1)
        def _():
            edge_loop(x1_hbm)

        plsc.subcore_barrier()

        pltpu.sync_copy(blk, cnt_sp.at[riota], add=True)
        plsc.subcore_barrier()

        def copy_out(dref):
            for n, (o, l) in enumerate(OFFS):
                buf = rows if n % 2 == 0 else rows2
                if n >= 2:
                    po, pll = OFFS[n - 2]
                    pltpu.make_async_copy(
                        buf.at[pl.ds(0, pll)],
                        dref.at[pl.ds(s * NPT + po, pll)], semw).wait()
                pltpu.sync_copy(acc.at[pl.ds(s * NPT + o, l)],
                                buf.at[pl.ds(0, l)])
                pltpu.async_copy(buf.at[pl.ds(0, l)],
                                 dref.at[pl.ds(s * NPT + o, l)], semw)
            for n in (len(OFFS) - 2, len(OFFS) - 1):
                o, l = OFFS[n]
                buf = rows if n % 2 == 0 else rows2
                pltpu.make_async_copy(buf.at[pl.ds(0, l)],
                                      dref.at[pl.ds(s * NPT + o, l)],
                                      semw).wait()

        @pl.when(c == 0)
        def _():
            copy_out(out0)

        @pl.when(c == 1)
        def _():
            copy_out(out1)

        @pl.when(jnp.logical_and(c == 0, s == 0))
        def _():
            pltpu.sync_copy(cnt_sp, blk)
            pltpu.sync_copy(blk, cnt_out)

    return k(x0, x1, src2, dst2, dst1)


BM = 400


def _tc_body(x_ref, nb0_ref, nb1_ref, cnt_ref, wst_ref, wnt0_ref, wnt1_ref,
             b_ref, g_ref, be_ref, o_ref):
    hs = jnp.dot(x_ref[...], wst_ref[...], preferred_element_type=jnp.float32)
    hn = (jnp.dot(nb0_ref[...], wnt0_ref[...], preferred_element_type=jnp.float32)
          + jnp.dot(nb1_ref[...], wnt1_ref[...], preferred_element_type=jnp.float32))
    inv = 1.0 / jnp.maximum(cnt_ref[...], 1.0)
    h = hs + hn * inv + b_ref[...]
    mu = jnp.mean(h, axis=-1, keepdims=True)
    d = h - mu
    var = jnp.mean(d * d, axis=-1, keepdims=True)
    o_ref[...] = d * lax.rsqrt(var + 1e-5) * g_ref[...] + be_ref[...]


def _tc_combine(x, nb0, nb1, cnt, wst, wnt0, wnt1, bias, gamma, beta):
    grid = (N_NODES // BM,)
    return pl.pallas_call(
        _tc_body,
        grid=grid,
        in_specs=[
            pl.BlockSpec((BM, D), lambda i: (i, 0)),
            pl.BlockSpec((BM, DH), lambda i: (i, 0)),
            pl.BlockSpec((BM, DH), lambda i: (i, 0)),
            pl.BlockSpec((BM, 1), lambda i: (i, 0)),
            pl.BlockSpec((D, D), lambda i: (0, 0)),
            pl.BlockSpec((DH, D), lambda i: (0, 0)),
            pl.BlockSpec((DH, D), lambda i: (0, 0)),
            pl.BlockSpec((1, D), lambda i: (0, 0)),
            pl.BlockSpec((1, D), lambda i: (0, 0)),
            pl.BlockSpec((1, D), lambda i: (0, 0)),
        ],
        out_specs=pl.BlockSpec((BM, D), lambda i: (i, 0)),
        out_shape=jax.ShapeDtypeStruct((N_NODES, D), jnp.float32),
    )(x, nb0, nb1, cnt, wst, wnt0, wnt1, bias, gamma, beta)


@jax.jit
def kernel(x, edge_index, deg, W_self, W_neigh, bias, ln_gamma, ln_beta):
    del deg
    x0 = x[:, :DH]
    x1 = x[:, DH:]
    src2 = edge_index[1].reshape(ROWS, K)
    dst2 = edge_index[0].reshape(ROWS, K)
    dst1 = edge_index[0]
    nb0, nb1, cnt_tab = _sc_segment_sum(x0, x1, src2, dst2, dst1)
    cnt = cnt_tab.reshape(NPAD)[:N_NODES, None]
    wnt = W_neigh.T
    return _tc_combine(x, nb0, nb1, cnt, W_self.T, wnt[:DH], wnt[DH:],
                       bias[None, :], ln_gamma[None, :], ln_beta[None, :])


# speedup vs baseline: 1.1154x; 1.0546x over previous
import functools

import jax
import jax.numpy as jnp
from jax import lax
from jax.experimental import pallas as pl
from jax.experimental.pallas import tpu as pltpu
from jax.experimental.pallas import tpu_sc as plsc

N_NODES = 10000
NPAD = 10240
D = 256
DH = 128
E = 160000
K = 100
ROWS = E // K
NS = 16
TROWS = ROWS // NS
EPT = E // NS
NPT = N_NODES // NS
G = 20
NGRP = TROWS // G
CG = G * K
CROWS = NPAD // DH
OFFS = ((0, 100), (100, 100), (200, 100), (300, 100), (400, 100),
        (500, 100), (600, 25))


def _sc_segment_sum(x0, x1, src2, dst2, dst1):
    mesh = plsc.VectorSubcoreMesh(core_axis_name="c", subcore_axis_name="s")

    @functools.partial(
        pl.kernel,
        mesh=mesh,
        compiler_params=pltpu.CompilerParams(use_tc_tiling_on_sc=False,
                                             needs_layout_passes=False),
        out_type=(
            jax.ShapeDtypeStruct((N_NODES, DH), jnp.float32),
            jax.ShapeDtypeStruct((N_NODES, DH), jnp.float32),
            jax.ShapeDtypeStruct((CROWS, DH), jnp.float32),
        ),
        scratch_types=[
            pltpu.VMEM((G, K), jnp.int32),
            pltpu.VMEM((G, K), jnp.int32),
            pltpu.VMEM((G, K), jnp.int32),
            pltpu.VMEM((G, K), jnp.int32),
            pltpu.VMEM((CG,), jnp.int32),
            pltpu.VMEM((CG,), jnp.int32),
            pltpu.VMEM((K, DH), jnp.float32),
            pltpu.VMEM((K, DH), jnp.float32),
            pltpu.VMEM((CROWS, DH), jnp.float32),
            pltpu.VMEM((CROWS,), jnp.int32),
            pltpu.VMEM_SHARED((N_NODES, DH), jnp.float32),
            pltpu.VMEM_SHARED((CROWS, DH), jnp.float32),
            pltpu.SemaphoreType.DMA,
            pltpu.SemaphoreType.DMA,
            pltpu.SemaphoreType.DMA,
            pltpu.SemaphoreType.DMA,
        ],
    )
    def k(x0_hbm, x1_hbm, src_hbm, dst_hbm, dst1_hbm, out0, out1, cnt_out,
          sidxa, sidxb, didxa, didxb, dcha, dchb, rows, rows2, blk, riota,
          acc, cnt_sp, sem, sem2, semi, semw):
        c = lax.axis_index("c")
        s = lax.axis_index("s")

        zero16 = jnp.zeros((16,), jnp.float32)
        one16 = jnp.ones((16,), jnp.float32)

        def zrow(i, carry):
            for j in range(DH // 16):
                rows[i, pl.ds(j * 16, 16)] = zero16
            return carry

        lax.fori_loop(0, K, zrow, 0)

        def brow(i, carry):
            for j in range(DH // 16):
                blk[i, pl.ds(j * 16, 16)] = zero16
            return carry

        lax.fori_loop(0, CROWS, brow, 0)

        for j in range(CROWS // 16):
            riota[pl.ds(j * 16, 16)] = lax.iota(jnp.int32, 16) + j * 16

        for o, l in OFFS:
            pltpu.async_copy(rows.at[pl.ds(0, l)],
                             acc.at[pl.ds(s * NPT + o, l)], semw)

        @pl.when(s == 0)
        def _():
            pltpu.async_copy(rows.at[pl.ds(0, CROWS)], cnt_sp, semw)

        def load_idx(g, sb, db, cb):
            pltpu.sync_copy(src_hbm.at[pl.ds(s * TROWS + g * G, G)], sb)
            pltpu.sync_copy(dst_hbm.at[pl.ds(s * TROWS + g * G, G)], db)
            pltpu.sync_copy(dst1_hbm.at[pl.ds(s * EPT + g * CG, CG)], cb)

        def prefetch_idx(g, sb, db, cb):
            pltpu.async_copy(src_hbm.at[pl.ds(s * TROWS + g * G, G)], sb, semi)
            pltpu.async_copy(dst_hbm.at[pl.ds(s * TROWS + g * G, G)], db, semi)
            pltpu.async_copy(dst1_hbm.at[pl.ds(s * EPT + g * CG, CG)], cb,
                             semi)

        def wait_idx(g, sb, db, cb):
            pltpu.make_async_copy(src_hbm.at[pl.ds(s * TROWS + g * G, G)], sb,
                                  semi).wait()
            pltpu.make_async_copy(dst_hbm.at[pl.ds(s * TROWS + g * G, G)], db,
                                  semi).wait()
            pltpu.make_async_copy(dst1_hbm.at[pl.ds(s * EPT + g * CG, CG)],
                                  cb, semi).wait()

        load_idx(0, sidxa, didxa, dcha)

        for o, l in OFFS:
            pltpu.make_async_copy(rows.at[pl.ds(0, l)],
                                  acc.at[pl.ds(s * NPT + o, l)], semw).wait()

        @pl.when(s == 0)
        def _():
            pltpu.make_async_copy(rows.at[pl.ds(0, CROWS)], cnt_sp,
                                  semw).wait()

        plsc.subcore_barrier()

        def edge_loop(x_ref):
            pltpu.async_copy(x_ref.at[sidxa.at[0]], rows, sem)
            bufs = (sidxa, didxa, dcha), (sidxb, didxb, dchb)
            for g in range(NGRP):
                cs, cd, cc = bufs[g % 2]
                ns, nd, nc = bufs[(g + 1) % 2]
                if g + 1 < NGRP:
                    prefetch_idx(g + 1, ns, nd, nc)

                def cbody(j, carry2):
                    idx = cc[pl.ds(j * 16, 16)]
                    plsc.addupdate_scatter(
                        blk,
                        [lax.shift_right_logical(idx, 7),
                         lax.bitwise_and(idx, 127)],
                        one16,
                    )
                    return carry2

                lax.fori_loop(0, CG // 16, cbody, 0)

                def pair(p, carry2):
                    pltpu.async_copy(x_ref.at[cs.at[2 * p + 1]], rows2, sem2)
                    pltpu.make_async_copy(x_ref.at[cs.at[2 * p]], rows,
                                          sem).wait()
                    pltpu.sync_copy(rows, acc.at[cd.at[2 * p]], add=True)

                    @pl.when(p < G // 2 - 1)
                    def _():
                        pltpu.async_copy(x_ref.at[cs.at[2 * p + 2]], rows,
                                         sem)

                    pltpu.make_async_copy(x_ref.at[cs.at[2 * p + 1]], rows2,
                                          sem2).wait()
                    pltpu.sync_copy(rows2, acc.at[cd.at[2 * p + 1]],
                                    add=True)
                    return carry2

                lax.fori_loop(0, G // 2, pair, 0)

                if g + 1 < NGRP:
                    wait_idx(g + 1, ns, nd, nc)
                    pltpu.async_copy(x_ref.at[ns.at[0]], rows, sem)

        @pl.when(c == 0)
        def _():
            edge_loop(x0_hbm)

        @pl.when(c == 1)
        def _():
            edge_loop(x1_hbm)

        plsc.subcore_barrier()

        pltpu.sync_copy(blk, cnt_sp.at[riota], add=True)
        plsc.subcore_barrier()

        def copy_out(dref):
            for n, (o, l) in enumerate(OFFS):
                buf = rows if n % 2 == 0 else rows2
                if n >= 2:
                    po, pll = OFFS[n - 2]
                    pltpu.make_async_copy(
                        buf.at[pl.ds(0, pll)],
                        dref.at[pl.ds(s * NPT + po, pll)], semw).wait()
                pltpu.sync_copy(acc.at[pl.ds(s * NPT + o, l)],
                                buf.at[pl.ds(0, l)])
                pltpu.async_copy(buf.at[pl.ds(0, l)],
                                 dref.at[pl.ds(s * NPT + o, l)], semw)
            for n in (len(OFFS) - 2, len(OFFS) - 1):
                o, l = OFFS[n]
                buf = rows if n % 2 == 0 else rows2
                pltpu.make_async_copy(buf.at[pl.ds(0, l)],
                                      dref.at[pl.ds(s * NPT + o, l)],
                                      semw).wait()

        @pl.when(c == 0)
        def _():
            copy_out(out0)

        @pl.when(c == 1)
        def _():
            copy_out(out1)

        @pl.when(jnp.logical_and(c == 0, s == 0))
        def _():
            pltpu.sync_copy(cnt_sp, blk)
            pltpu.sync_copy(blk, cnt_out)

    return k(x0, x1, src2, dst2, dst1)


BM = 1000


def _tc_body(x_ref, nb0_ref, nb1_ref, cnt_ref, wst_ref, wnt0_ref, wnt1_ref,
             b_ref, g_ref, be_ref, o_ref):
    hs = jnp.dot(x_ref[...], wst_ref[...], preferred_element_type=jnp.float32)
    hn = (jnp.dot(nb0_ref[...], wnt0_ref[...], preferred_element_type=jnp.float32)
          + jnp.dot(nb1_ref[...], wnt1_ref[...], preferred_element_type=jnp.float32))
    inv = 1.0 / jnp.maximum(cnt_ref[...], 1.0)
    h = hs + hn * inv + b_ref[...]
    mu = jnp.mean(h, axis=-1, keepdims=True)
    d = h - mu
    var = jnp.mean(d * d, axis=-1, keepdims=True)
    o_ref[...] = d * lax.rsqrt(var + 1e-5) * g_ref[...] + be_ref[...]


def _tc_combine(x, nb0, nb1, cnt, wst, wnt0, wnt1, bias, gamma, beta):
    grid = (N_NODES // BM,)
    return pl.pallas_call(
        _tc_body,
        grid=grid,
        in_specs=[
            pl.BlockSpec((BM, D), lambda i: (i, 0)),
            pl.BlockSpec((BM, DH), lambda i: (i, 0)),
            pl.BlockSpec((BM, DH), lambda i: (i, 0)),
            pl.BlockSpec((BM, 1), lambda i: (i, 0)),
            pl.BlockSpec((D, D), lambda i: (0, 0)),
            pl.BlockSpec((DH, D), lambda i: (0, 0)),
            pl.BlockSpec((DH, D), lambda i: (0, 0)),
            pl.BlockSpec((1, D), lambda i: (0, 0)),
            pl.BlockSpec((1, D), lambda i: (0, 0)),
            pl.BlockSpec((1, D), lambda i: (0, 0)),
        ],
        out_specs=pl.BlockSpec((BM, D), lambda i: (i, 0)),
        out_shape=jax.ShapeDtypeStruct((N_NODES, D), jnp.float32),
    )(x, nb0, nb1, cnt, wst, wnt0, wnt1, bias, gamma, beta)


@jax.jit
def kernel(x, edge_index, deg, W_self, W_neigh, bias, ln_gamma, ln_beta):
    del deg
    x0 = x[:, :DH]
    x1 = x[:, DH:]
    src2 = edge_index[1].reshape(ROWS, K)
    dst2 = edge_index[0].reshape(ROWS, K)
    dst1 = edge_index[0]
    nb0, nb1, cnt_tab = _sc_segment_sum(x0, x1, src2, dst2, dst1)
    cnt = cnt_tab.reshape(NPAD)[:N_NODES, None]
    wnt = W_neigh.T
    return _tc_combine(x, nb0, nb1, cnt, W_self.T, wnt[:DH], wnt[DH:],
                       bias[None, :], ln_gamma[None, :], ln_beta[None, :])


# speedup vs baseline: 1.1224x; 1.0062x over previous
import functools

import jax
import jax.numpy as jnp
from jax import lax
from jax.experimental import pallas as pl
from jax.experimental.pallas import tpu as pltpu
from jax.experimental.pallas import tpu_sc as plsc

N_NODES = 10000
NPAD = 10240
D = 256
DH = 128
E = 160000
K = 100
ROWS = E // K
NS = 16
TROWS = ROWS // NS
EPT = E // NS
NPT = N_NODES // NS
G = 20
NGRP = TROWS // G
CG = G * K
CROWS = NPAD // DH
OFFS = ((0, 100), (100, 100), (200, 100), (300, 100), (400, 100),
        (500, 100), (600, 25))


def _sc_segment_sum(x0, x1, src2, dst2, dst1):
    mesh = plsc.VectorSubcoreMesh(core_axis_name="c", subcore_axis_name="s")

    @functools.partial(
        pl.kernel,
        mesh=mesh,
        compiler_params=pltpu.CompilerParams(use_tc_tiling_on_sc=False,
                                             needs_layout_passes=False),
        out_type=(
            jax.ShapeDtypeStruct((N_NODES, DH), jnp.float32),
            jax.ShapeDtypeStruct((N_NODES, DH), jnp.float32),
            jax.ShapeDtypeStruct((CROWS, DH), jnp.float32),
        ),
        scratch_types=[
            pltpu.VMEM((G, K), jnp.int32),
            pltpu.VMEM((G, K), jnp.int32),
            pltpu.VMEM((G, K), jnp.int32),
            pltpu.VMEM((G, K), jnp.int32),
            pltpu.VMEM((CG,), jnp.int32),
            pltpu.VMEM((CG,), jnp.int32),
            pltpu.VMEM((K, DH), jnp.float32),
            pltpu.VMEM((K, DH), jnp.float32),
            pltpu.VMEM((CROWS, DH), jnp.float32),
            pltpu.VMEM((CROWS,), jnp.int32),
            pltpu.VMEM_SHARED((N_NODES, DH), jnp.float32),
            pltpu.VMEM_SHARED((CROWS, DH), jnp.float32),
            pltpu.SemaphoreType.DMA,
            pltpu.SemaphoreType.DMA,
            pltpu.SemaphoreType.DMA,
            pltpu.SemaphoreType.DMA,
        ],
    )
    def k(x0_hbm, x1_hbm, src_hbm, dst_hbm, dst1_hbm, out0, out1, cnt_out,
          sidxa, sidxb, didxa, didxb, dcha, dchb, rows, rows2, blk, riota,
          acc, cnt_sp, sem, sem2, semi, semw):
        c = lax.axis_index("c")
        s = lax.axis_index("s")

        zero16 = jnp.zeros((16,), jnp.float32)
        one16 = jnp.ones((16,), jnp.float32)

        def zrow(i, carry):
            for j in range(DH // 16):
                rows[i, pl.ds(j * 16, 16)] = zero16
            return carry

        lax.fori_loop(0, K, zrow, 0)

        def brow(i, carry):
            for j in range(DH // 16):
                blk[i, pl.ds(j * 16, 16)] = zero16
            return carry

        lax.fori_loop(0, CROWS, brow, 0)

        for j in range(CROWS // 16):
            riota[pl.ds(j * 16, 16)] = lax.iota(jnp.int32, 16) + j * 16

        for o, l in OFFS:
            pltpu.async_copy(rows.at[pl.ds(0, l)],
                             acc.at[pl.ds(s * NPT + o, l)], semw)

        @pl.when(s == 0)
        def _():
            pltpu.async_copy(rows.at[pl.ds(0, CROWS)], cnt_sp, semw)

        def load_idx(g, sb, db, cb):
            pltpu.sync_copy(src_hbm.at[pl.ds(s * TROWS + g * G, G)], sb)
            pltpu.sync_copy(dst_hbm.at[pl.ds(s * TROWS + g * G, G)], db)
            pltpu.sync_copy(dst1_hbm.at[pl.ds(s * EPT + g * CG, CG)], cb)

        def prefetch_idx(g, sb, db, cb):
            pltpu.async_copy(src_hbm.at[pl.ds(s * TROWS + g * G, G)], sb, semi)
            pltpu.async_copy(dst_hbm.at[pl.ds(s * TROWS + g * G, G)], db, semi)
            pltpu.async_copy(dst1_hbm.at[pl.ds(s * EPT + g * CG, CG)], cb,
                             semi)

        def wait_idx(g, sb, db, cb):
            pltpu.make_async_copy(src_hbm.at[pl.ds(s * TROWS + g * G, G)], sb,
                                  semi).wait()
            pltpu.make_async_copy(dst_hbm.at[pl.ds(s * TROWS + g * G, G)], db,
                                  semi).wait()
            pltpu.make_async_copy(dst1_hbm.at[pl.ds(s * EPT + g * CG, CG)],
                                  cb, semi).wait()

        load_idx(0, sidxa, didxa, dcha)

        for o, l in OFFS:
            pltpu.make_async_copy(rows.at[pl.ds(0, l)],
                                  acc.at[pl.ds(s * NPT + o, l)], semw).wait()

        @pl.when(s == 0)
        def _():
            pltpu.make_async_copy(rows.at[pl.ds(0, CROWS)], cnt_sp,
                                  semw).wait()

        plsc.subcore_barrier()

        def edge_loop(x_ref):
            pltpu.async_copy(x_ref.at[sidxa.at[0]], rows, sem)
            bufs = (sidxa, didxa, dcha), (sidxb, didxb, dchb)
            for g in range(NGRP):
                cs, cd, cc = bufs[g % 2]
                ns, nd, nc = bufs[(g + 1) % 2]
                if g + 1 < NGRP:
                    prefetch_idx(g + 1, ns, nd, nc)

                def cbody(j, carry2):
                    idx = cc[pl.ds(j * 16, 16)]
                    plsc.addupdate_scatter(
                        blk,
                        [lax.shift_right_logical(idx, 7),
                         lax.bitwise_and(idx, 127)],
                        one16,
                    )
                    return carry2

                lax.fori_loop(0, CG // 16, cbody, 0)

                def pair(p, carry2):
                    pltpu.async_copy(x_ref.at[cs.at[2 * p + 1]], rows2, sem2)
                    pltpu.make_async_copy(x_ref.at[cs.at[2 * p]], rows,
                                          sem).wait()
                    pltpu.sync_copy(rows, acc.at[cd.at[2 * p]], add=True)

                    @pl.when(p < G // 2 - 1)
                    def _():
                        pltpu.async_copy(x_ref.at[cs.at[2 * p + 2]], rows,
                                         sem)

                    pltpu.make_async_copy(x_ref.at[cs.at[2 * p + 1]], rows2,
                                          sem2).wait()
                    pltpu.sync_copy(rows2, acc.at[cd.at[2 * p + 1]],
                                    add=True)
                    return carry2

                lax.fori_loop(0, G // 2, pair, 0)

                if g + 1 < NGRP:
                    wait_idx(g + 1, ns, nd, nc)
                    pltpu.async_copy(x_ref.at[ns.at[0]], rows, sem)

        @pl.when(c == 0)
        def _():
            edge_loop(x0_hbm)

        @pl.when(c == 1)
        def _():
            edge_loop(x1_hbm)

        plsc.subcore_barrier()

        pltpu.sync_copy(blk, cnt_sp.at[riota], add=True)
        plsc.subcore_barrier()

        def copy_out(dref):
            for n, (o, l) in enumerate(OFFS):
                buf = rows if n % 2 == 0 else rows2
                if n >= 2:
                    po, pll = OFFS[n - 2]
                    pltpu.make_async_copy(
                        buf.at[pl.ds(0, pll)],
                        dref.at[pl.ds(s * NPT + po, pll)], semw).wait()
                pltpu.sync_copy(acc.at[pl.ds(s * NPT + o, l)],
                                buf.at[pl.ds(0, l)])
                pltpu.async_copy(buf.at[pl.ds(0, l)],
                                 dref.at[pl.ds(s * NPT + o, l)], semw)
            for n in (len(OFFS) - 2, len(OFFS) - 1):
                o, l = OFFS[n]
                buf = rows if n % 2 == 0 else rows2
                pltpu.make_async_copy(buf.at[pl.ds(0, l)],
                                      dref.at[pl.ds(s * NPT + o, l)],
                                      semw).wait()

        @pl.when(c == 0)
        def _():
            copy_out(out0)

        @pl.when(c == 1)
        def _():
            copy_out(out1)

        @pl.when(jnp.logical_and(c == 0, s == 0))
        def _():
            pltpu.sync_copy(cnt_sp, blk)
            pltpu.sync_copy(blk, cnt_out)

    return k(x0, x1, src2, dst2, dst1)


BM = 2000


def _tc_body(x_ref, nb0_ref, nb1_ref, cnt_ref, wst_ref, wnt0_ref, wnt1_ref,
             b_ref, g_ref, be_ref, o_ref):
    hs = jnp.dot(x_ref[...], wst_ref[...], preferred_element_type=jnp.float32)
    hn = (jnp.dot(nb0_ref[...], wnt0_ref[...], preferred_element_type=jnp.float32)
          + jnp.dot(nb1_ref[...], wnt1_ref[...], preferred_element_type=jnp.float32))
    inv = 1.0 / jnp.maximum(cnt_ref[...], 1.0)
    h = hs + hn * inv + b_ref[...]
    mu = jnp.mean(h, axis=-1, keepdims=True)
    d = h - mu
    var = jnp.mean(d * d, axis=-1, keepdims=True)
    o_ref[...] = d * lax.rsqrt(var + 1e-5) * g_ref[...] + be_ref[...]


def _tc_combine(x, nb0, nb1, cnt, wst, wnt0, wnt1, bias, gamma, beta):
    grid = (N_NODES // BM,)
    return pl.pallas_call(
        _tc_body,
        grid=grid,
        in_specs=[
            pl.BlockSpec((BM, D), lambda i: (i, 0)),
            pl.BlockSpec((BM, DH), lambda i: (i, 0)),
            pl.BlockSpec((BM, DH), lambda i: (i, 0)),
            pl.BlockSpec((BM, 1), lambda i: (i, 0)),
            pl.BlockSpec((D, D), lambda i: (0, 0)),
            pl.BlockSpec((DH, D), lambda i: (0, 0)),
            pl.BlockSpec((DH, D), lambda i: (0, 0)),
            pl.BlockSpec((1, D), lambda i: (0, 0)),
            pl.BlockSpec((1, D), lambda i: (0, 0)),
            pl.BlockSpec((1, D), lambda i: (0, 0)),
        ],
        out_specs=pl.BlockSpec((BM, D), lambda i: (i, 0)),
        out_shape=jax.ShapeDtypeStruct((N_NODES, D), jnp.float32),
    )(x, nb0, nb1, cnt, wst, wnt0, wnt1, bias, gamma, beta)


@jax.jit
def kernel(x, edge_index, deg, W_self, W_neigh, bias, ln_gamma, ln_beta):
    del deg
    x0 = x[:, :DH]
    x1 = x[:, DH:]
    src2 = edge_index[1].reshape(ROWS, K)
    dst2 = edge_index[0].reshape(ROWS, K)
    dst1 = edge_index[0]
    nb0, nb1, cnt_tab = _sc_segment_sum(x0, x1, src2, dst2, dst1)
    cnt = cnt_tab.reshape(NPAD)[:N_NODES, None]
    wnt = W_neigh.T
    return _tc_combine(x, nb0, nb1, cnt, W_self.T, wnt[:DH], wnt[DH:],
                       bias[None, :], ln_gamma[None, :], ln_beta[None, :])


# speedup vs baseline: 1.1655x; 1.0384x over previous
import functools

import jax
import jax.numpy as jnp
from jax import lax
from jax.experimental import pallas as pl
from jax.experimental.pallas import tpu as pltpu
from jax.experimental.pallas import tpu_sc as plsc

N_NODES = 10000
NPAD = 10240
D = 256
DH = 128
E = 160000
K = 100
ROWS = E // K
NS = 16
TROWS = ROWS // NS
EPT = E // NS
NPT = N_NODES // NS
G = 20
NGRP = TROWS // G
CG = G * K
CROWS = NPAD // DH
OFFS = ((0, 100), (100, 100), (200, 100), (300, 100), (400, 100),
        (500, 100), (600, 25))


def _sc_segment_sum(x0, x1, src2, dst2, dst1):
    mesh = plsc.VectorSubcoreMesh(core_axis_name="c", subcore_axis_name="s")

    @functools.partial(
        pl.kernel,
        mesh=mesh,
        compiler_params=pltpu.CompilerParams(use_tc_tiling_on_sc=False,
                                             needs_layout_passes=False),
        out_type=(
            jax.ShapeDtypeStruct((N_NODES, DH), jnp.bfloat16),
            jax.ShapeDtypeStruct((N_NODES, DH), jnp.bfloat16),
            jax.ShapeDtypeStruct((CROWS, DH), jnp.float32),
        ),
        scratch_types=[
            pltpu.VMEM((G, K), jnp.int32),
            pltpu.VMEM((G, K), jnp.int32),
            pltpu.VMEM((G, K), jnp.int32),
            pltpu.VMEM((G, K), jnp.int32),
            pltpu.VMEM((CG,), jnp.int32),
            pltpu.VMEM((CG,), jnp.int32),
            pltpu.VMEM((K, DH), jnp.bfloat16),
            pltpu.VMEM((K, DH), jnp.bfloat16),
            pltpu.VMEM((CROWS, DH), jnp.float32),
            pltpu.VMEM((CROWS,), jnp.int32),
            pltpu.VMEM_SHARED((N_NODES, DH), jnp.bfloat16),
            pltpu.VMEM_SHARED((CROWS, DH), jnp.float32),
            pltpu.SemaphoreType.DMA,
            pltpu.SemaphoreType.DMA,
            pltpu.SemaphoreType.DMA,
            pltpu.SemaphoreType.DMA,
        ],
    )
    def k(x0_hbm, x1_hbm, src_hbm, dst_hbm, dst1_hbm, out0, out1, cnt_out,
          sidxa, sidxb, didxa, didxb, dcha, dchb, rows, rows2, blk, riota,
          acc, cnt_sp, sem, sem2, semi, semw):
        c = lax.axis_index("c")
        s = lax.axis_index("s")

        zero16 = jnp.zeros((16,), jnp.float32)
        one16 = jnp.ones((16,), jnp.float32)
        zero32 = jnp.zeros((32,), jnp.bfloat16)

        def zrow(i, carry):
            for j in range(DH // 32):
                rows[i, pl.ds(j * 32, 32)] = zero32
            return carry

        lax.fori_loop(0, K, zrow, 0)

        def brow(i, carry):
            for j in range(DH // 16):
                blk[i, pl.ds(j * 16, 16)] = zero16
            return carry

        lax.fori_loop(0, CROWS, brow, 0)

        for j in range(CROWS // 16):
            riota[pl.ds(j * 16, 16)] = lax.iota(jnp.int32, 16) + j * 16

        for o, l in OFFS:
            pltpu.async_copy(rows.at[pl.ds(0, l)],
                             acc.at[pl.ds(s * NPT + o, l)], semw)

        @pl.when(s == 0)
        def _():
            pltpu.async_copy(blk, cnt_sp, semw)

        def load_idx(g, sb, db, cb):
            pltpu.sync_copy(src_hbm.at[pl.ds(s * TROWS + g * G, G)], sb)
            pltpu.sync_copy(dst_hbm.at[pl.ds(s * TROWS + g * G, G)], db)
            pltpu.sync_copy(dst1_hbm.at[pl.ds(s * EPT + g * CG, CG)], cb)

        def prefetch_idx(g, sb, db, cb):
            pltpu.async_copy(src_hbm.at[pl.ds(s * TROWS + g * G, G)], sb, semi)
            pltpu.async_copy(dst_hbm.at[pl.ds(s * TROWS + g * G, G)], db, semi)
            pltpu.async_copy(dst1_hbm.at[pl.ds(s * EPT + g * CG, CG)], cb,
                             semi)

        def wait_idx(g, sb, db, cb):
            pltpu.make_async_copy(src_hbm.at[pl.ds(s * TROWS + g * G, G)], sb,
                                  semi).wait()
            pltpu.make_async_copy(dst_hbm.at[pl.ds(s * TROWS + g * G, G)], db,
                                  semi).wait()
            pltpu.make_async_copy(dst1_hbm.at[pl.ds(s * EPT + g * CG, CG)],
                                  cb, semi).wait()

        load_idx(0, sidxa, didxa, dcha)

        for o, l in OFFS:
            pltpu.make_async_copy(rows.at[pl.ds(0, l)],
                                  acc.at[pl.ds(s * NPT + o, l)], semw).wait()

        @pl.when(s == 0)
        def _():
            pltpu.make_async_copy(blk, cnt_sp, semw).wait()

        plsc.subcore_barrier()

        def edge_loop(x_ref):
            pltpu.async_copy(x_ref.at[sidxa.at[0]], rows, sem)
            bufs = (sidxa, didxa, dcha), (sidxb, didxb, dchb)
            for g in range(NGRP):
                cs, cd, cc = bufs[g % 2]
                ns, nd, nc = bufs[(g + 1) % 2]
                if g + 1 < NGRP:
                    prefetch_idx(g + 1, ns, nd, nc)

                def cbody(j, carry2):
                    idx = cc[pl.ds(j * 16, 16)]
                    plsc.addupdate_scatter(
                        blk,
                        [lax.shift_right_logical(idx, 7),
                         lax.bitwise_and(idx, 127)],
                        one16,
                    )
                    return carry2

                lax.fori_loop(0, CG // 16, cbody, 0)

                def pair(p, carry2):
                    pltpu.async_copy(x_ref.at[cs.at[2 * p + 1]], rows2, sem2)
                    pltpu.make_async_copy(x_ref.at[cs.at[2 * p]], rows,
                                          sem).wait()
                    pltpu.sync_copy(rows, acc.at[cd.at[2 * p]], add=True)

                    @pl.when(p < G // 2 - 1)
                    def _():
                        pltpu.async_copy(x_ref.at[cs.at[2 * p + 2]], rows,
                                         sem)

                    pltpu.make_async_copy(x_ref.at[cs.at[2 * p + 1]], rows2,
                                          sem2).wait()
                    pltpu.sync_copy(rows2, acc.at[cd.at[2 * p + 1]],
                                    add=True)
                    return carry2

                lax.fori_loop(0, G // 2, pair, 0)

                if g + 1 < NGRP:
                    wait_idx(g + 1, ns, nd, nc)
                    pltpu.async_copy(x_ref.at[ns.at[0]], rows, sem)

        @pl.when(c == 0)
        def _():
            edge_loop(x0_hbm)

        @pl.when(c == 1)
        def _():
            edge_loop(x1_hbm)

        plsc.subcore_barrier()

        pltpu.sync_copy(blk, cnt_sp.at[riota], add=True)
        plsc.subcore_barrier()

        def copy_out(dref):
            for n, (o, l) in enumerate(OFFS):
                buf = rows if n % 2 == 0 else rows2
                if n >= 2:
                    po, pll = OFFS[n - 2]
                    pltpu.make_async_copy(
                        buf.at[pl.ds(0, pll)],
                        dref.at[pl.ds(s * NPT + po, pll)], semw).wait()
                pltpu.sync_copy(acc.at[pl.ds(s * NPT + o, l)],
                                buf.at[pl.ds(0, l)])
                pltpu.async_copy(buf.at[pl.ds(0, l)],
                                 dref.at[pl.ds(s * NPT + o, l)], semw)
            for n in (len(OFFS) - 2, len(OFFS) - 1):
                o, l = OFFS[n]
                buf = rows if n % 2 == 0 else rows2
                pltpu.make_async_copy(buf.at[pl.ds(0, l)],
                                      dref.at[pl.ds(s * NPT + o, l)],
                                      semw).wait()

        @pl.when(c == 0)
        def _():
            copy_out(out0)

        @pl.when(c == 1)
        def _():
            copy_out(out1)

        @pl.when(jnp.logical_and(c == 0, s == 0))
        def _():
            pltpu.sync_copy(cnt_sp, blk)
            pltpu.sync_copy(blk, cnt_out)

    return k(x0, x1, src2, dst2, dst1)


BM = 2000


def _tc_body(x_ref, nb0_ref, nb1_ref, cnt_ref, wst_ref, wnt0_ref, wnt1_ref,
             b_ref, g_ref, be_ref, o_ref):
    hs = jnp.dot(x_ref[...], wst_ref[...], preferred_element_type=jnp.float32)
    hn = (jnp.dot(nb0_ref[...].astype(jnp.float32), wnt0_ref[...],
                  preferred_element_type=jnp.float32)
          + jnp.dot(nb1_ref[...].astype(jnp.float32), wnt1_ref[...],
                    preferred_element_type=jnp.float32))
    inv = 1.0 / jnp.maximum(cnt_ref[...], 1.0)
    h = hs + hn * inv + b_ref[...]
    mu = jnp.mean(h, axis=-1, keepdims=True)
    d = h - mu
    var = jnp.mean(d * d, axis=-1, keepdims=True)
    o_ref[...] = d * lax.rsqrt(var + 1e-5) * g_ref[...] + be_ref[...]


def _tc_combine(x, nb0, nb1, cnt, wst, wnt0, wnt1, bias, gamma, beta):
    grid = (N_NODES // BM,)
    return pl.pallas_call(
        _tc_body,
        grid=grid,
        in_specs=[
            pl.BlockSpec((BM, D), lambda i: (i, 0)),
            pl.BlockSpec((BM, DH), lambda i: (i, 0)),
            pl.BlockSpec((BM, DH), lambda i: (i, 0)),
            pl.BlockSpec((BM, 1), lambda i: (i, 0)),
            pl.BlockSpec((D, D), lambda i: (0, 0)),
            pl.BlockSpec((DH, D), lambda i: (0, 0)),
            pl.BlockSpec((DH, D), lambda i: (0, 0)),
            pl.BlockSpec((1, D), lambda i: (0, 0)),
            pl.BlockSpec((1, D), lambda i: (0, 0)),
            pl.BlockSpec((1, D), lambda i: (0, 0)),
        ],
        out_specs=pl.BlockSpec((BM, D), lambda i: (i, 0)),
        out_shape=jax.ShapeDtypeStruct((N_NODES, D), jnp.float32),
    )(x, nb0, nb1, cnt, wst, wnt0, wnt1, bias, gamma, beta)


@jax.jit
def kernel(x, edge_index, deg, W_self, W_neigh, bias, ln_gamma, ln_beta):
    del deg
    x0 = x[:, :DH].astype(jnp.bfloat16)
    x1 = x[:, DH:].astype(jnp.bfloat16)
    src2 = edge_index[1].reshape(ROWS, K)
    dst2 = edge_index[0].reshape(ROWS, K)
    dst1 = edge_index[0]
    nb0, nb1, cnt_tab = _sc_segment_sum(x0, x1, src2, dst2, dst1)
    cnt = cnt_tab.reshape(NPAD)[:N_NODES, None]
    wnt = W_neigh.T
    return _tc_combine(x, nb0, nb1, cnt, W_self.T, wnt[:DH], wnt[DH:],
                       bias[None, :], ln_gamma[None, :], ln_beta[None, :])


# speedup vs baseline: 1.3308x; 1.1419x over previous
import functools

import jax
import jax.numpy as jnp
from jax import lax
from jax.experimental import pallas as pl
from jax.experimental.pallas import tpu as pltpu
from jax.experimental.pallas import tpu_sc as plsc

N_NODES = 10000
NPAD = 10240
D = 256
DH = 128
E = 160000
K = 100
ROWS = E // K
NS = 16
TROWS = ROWS // NS
EPT = E // NS
NPT = N_NODES // NS
NBUF = 5
CROWS = NPAD // DH
OFFS = ((0, 100), (100, 100), (200, 100), (300, 100), (400, 100),
        (500, 100), (600, 25))


def _sc_segment_sum(x0, x1, src2, dst2, dst1):
    mesh = plsc.VectorSubcoreMesh(core_axis_name="c", subcore_axis_name="s")

    @functools.partial(
        pl.kernel,
        mesh=mesh,
        compiler_params=pltpu.CompilerParams(use_tc_tiling_on_sc=False,
                                             needs_layout_passes=False),
        out_type=(
            jax.ShapeDtypeStruct((N_NODES, DH), jnp.bfloat16),
            jax.ShapeDtypeStruct((N_NODES, DH), jnp.bfloat16),
            jax.ShapeDtypeStruct((CROWS, DH), jnp.float32),
        ),
        scratch_types=[
            pltpu.VMEM((TROWS, K), jnp.int32),
            pltpu.VMEM((TROWS, K), jnp.int32),
            pltpu.VMEM((EPT,), jnp.int32),
            [pltpu.VMEM((K, DH), jnp.bfloat16) for _ in range(NBUF)],
            pltpu.VMEM((CROWS, DH), jnp.float32),
            pltpu.VMEM((CROWS,), jnp.int32),
            pltpu.VMEM_SHARED((N_NODES, DH), jnp.bfloat16),
            pltpu.VMEM_SHARED((CROWS, DH), jnp.float32),
            [pltpu.SemaphoreType.DMA for _ in range(NBUF)],
            [pltpu.SemaphoreType.DMA for _ in range(NBUF)],
            pltpu.SemaphoreType.DMA,
            pltpu.SemaphoreType.DMA,
        ],
    )
    def k(x0_hbm, x1_hbm, src_hbm, dst_hbm, dst1_hbm, out0, out1, cnt_out,
          sidx, didx, dch, bufs, blk, riota, acc, cnt_sp, semg, sems,
          semi, semw):
        c = lax.axis_index("c")
        s = lax.axis_index("s")

        zero16 = jnp.zeros((16,), jnp.float32)
        one16 = jnp.ones((16,), jnp.float32)
        zero32 = jnp.zeros((32,), jnp.bfloat16)

        pltpu.async_copy(src_hbm.at[pl.ds(s * TROWS, TROWS)], sidx, semi)
        pltpu.async_copy(dst_hbm.at[pl.ds(s * TROWS, TROWS)], didx, semi)
        pltpu.async_copy(dst1_hbm.at[pl.ds(s * EPT, EPT)], dch, semi)

        rows0 = bufs[0]

        def zrow(i, carry):
            for j in range(DH // 32):
                rows0[i, pl.ds(j * 32, 32)] = zero32
            return carry

        lax.fori_loop(0, K, zrow, 0)

        def brow(i, carry):
            for j in range(DH // 16):
                blk[i, pl.ds(j * 16, 16)] = zero16
            return carry

        lax.fori_loop(0, CROWS, brow, 0)

        for j in range(CROWS // 16):
            riota[pl.ds(j * 16, 16)] = lax.iota(jnp.int32, 16) + j * 16

        for o, l in OFFS:
            pltpu.async_copy(rows0.at[pl.ds(0, l)],
                             acc.at[pl.ds(s * NPT + o, l)], semw)

        @pl.when(s == 0)
        def _():
            pltpu.async_copy(blk, cnt_sp, semw)

        for o, l in OFFS:
            pltpu.make_async_copy(rows0.at[pl.ds(0, l)],
                                  acc.at[pl.ds(s * NPT + o, l)], semw).wait()

        @pl.when(s == 0)
        def _():
            pltpu.make_async_copy(blk, cnt_sp, semw).wait()

        pltpu.make_async_copy(src_hbm.at[pl.ds(s * TROWS, TROWS)], sidx,
                              semi).wait()
        pltpu.make_async_copy(dst_hbm.at[pl.ds(s * TROWS, TROWS)], didx,
                              semi).wait()
        pltpu.make_async_copy(dst1_hbm.at[pl.ds(s * EPT, EPT)], dch,
                              semi).wait()

        plsc.subcore_barrier()

        def edge_loop(x_ref):
            for i in range(NBUF):
                pltpu.async_copy(x_ref.at[sidx.at[i]], bufs[i], semg[i])

            def cbody(j, carry):
                idx = dch[pl.ds(j * 16, 16)]
                plsc.addupdate_scatter(
                    blk,
                    [lax.shift_right_logical(idx, 7),
                     lax.bitwise_and(idx, 127)],
                    one16,
                )
                return carry

            lax.fori_loop(0, EPT // 16, cbody, 0)

            def step(q, carry):
                for i in range(NBUF):
                    a = NBUF * q + i
                    pltpu.make_async_copy(x_ref.at[sidx.at[a]], bufs[i],
                                          semg[i]).wait()
                    pltpu.async_copy(bufs[i], acc.at[didx.at[a]], sems[i],
                                     add=True)
                    j = (i - 2) % NBUF
                    b = a - 2

                    @pl.when(jnp.logical_and(b >= 0, b + NBUF < TROWS))
                    def _():
                        pltpu.make_async_copy(bufs[j], acc.at[didx.at[b]],
                                              sems[j]).wait()
                        pltpu.async_copy(x_ref.at[sidx.at[b + NBUF]], bufs[j],
                                         semg[j])
                return carry

            lax.fori_loop(0, TROWS // NBUF, step, 0)

            for a in range(TROWS - NBUF, TROWS):
                i = a % NBUF
                pltpu.make_async_copy(bufs[i], acc.at[didx.at[a]],
                                      sems[i]).wait()

        @pl.when(c == 0)
        def _():
            edge_loop(x0_hbm)

        @pl.when(c == 1)
        def _():
            edge_loop(x1_hbm)

        plsc.subcore_barrier()

        pltpu.sync_copy(blk, cnt_sp.at[riota], add=True)
        plsc.subcore_barrier()

        def copy_out(dref):
            for n, (o, l) in enumerate(OFFS):
                buf = bufs[n % 2]
                if n >= 2:
                    po, pll = OFFS[n - 2]
                    pltpu.make_async_copy(
                        buf.at[pl.ds(0, pll)],
                        dref.at[pl.ds(s * NPT + po, pll)], semw).wait()
                pltpu.sync_copy(acc.at[pl.ds(s * NPT + o, l)],
                                buf.at[pl.ds(0, l)])
                pltpu.async_copy(buf.at[pl.ds(0, l)],
                                 dref.at[pl.ds(s * NPT + o, l)], semw)
            for n in (len(OFFS) - 2, len(OFFS) - 1):
                o, l = OFFS[n]
                buf = bufs[n % 2]
                pltpu.make_async_copy(buf.at[pl.ds(0, l)],
                                      dref.at[pl.ds(s * NPT + o, l)],
                                      semw).wait()

        @pl.when(c == 0)
        def _():
            copy_out(out0)

        @pl.when(c == 1)
        def _():
            copy_out(out1)

        @pl.when(jnp.logical_and(c == 0, s == 0))
        def _():
            pltpu.sync_copy(cnt_sp, blk)
            pltpu.sync_copy(blk, cnt_out)

    return k(x0, x1, src2, dst2, dst1)


BM = 2000


def _tc_body(x_ref, nb0_ref, nb1_ref, cnt_ref, wst_ref, wnt0_ref, wnt1_ref,
             b_ref, g_ref, be_ref, o_ref):
    hs = jnp.dot(x_ref[...], wst_ref[...], preferred_element_type=jnp.float32)
    hn = (jnp.dot(nb0_ref[...].astype(jnp.float32), wnt0_ref[...],
                  preferred_element_type=jnp.float32)
          + jnp.dot(nb1_ref[...].astype(jnp.float32), wnt1_ref[...],
                    preferred_element_type=jnp.float32))
    inv = 1.0 / jnp.maximum(cnt_ref[...], 1.0)
    h = hs + hn * inv + b_ref[...]
    mu = jnp.mean(h, axis=-1, keepdims=True)
    d = h - mu
    var = jnp.mean(d * d, axis=-1, keepdims=True)
    o_ref[...] = d * lax.rsqrt(var + 1e-5) * g_ref[...] + be_ref[...]


def _tc_combine(x, nb0, nb1, cnt, wst, wnt0, wnt1, bias, gamma, beta):
    grid = (N_NODES // BM,)
    return pl.pallas_call(
        _tc_body,
        grid=grid,
        in_specs=[
            pl.BlockSpec((BM, D), lambda i: (i, 0)),
            pl.BlockSpec((BM, DH), lambda i: (i, 0)),
            pl.BlockSpec((BM, DH), lambda i: (i, 0)),
            pl.BlockSpec((BM, 1), lambda i: (i, 0)),
            pl.BlockSpec((D, D), lambda i: (0, 0)),
            pl.BlockSpec((DH, D), lambda i: (0, 0)),
            pl.BlockSpec((DH, D), lambda i: (0, 0)),
            pl.BlockSpec((1, D), lambda i: (0, 0)),
            pl.BlockSpec((1, D), lambda i: (0, 0)),
            pl.BlockSpec((1, D), lambda i: (0, 0)),
        ],
        out_specs=pl.BlockSpec((BM, D), lambda i: (i, 0)),
        out_shape=jax.ShapeDtypeStruct((N_NODES, D), jnp.float32),
    )(x, nb0, nb1, cnt, wst, wnt0, wnt1, bias, gamma, beta)


@jax.jit
def kernel(x, edge_index, deg, W_self, W_neigh, bias, ln_gamma, ln_beta):
    del deg
    x0 = x[:, :DH].astype(jnp.bfloat16)
    x1 = x[:, DH:].astype(jnp.bfloat16)
    src2 = edge_index[1].reshape(ROWS, K)
    dst2 = edge_index[0].reshape(ROWS, K)
    dst1 = edge_index[0]
    nb0, nb1, cnt_tab = _sc_segment_sum(x0, x1, src2, dst2, dst1)
    cnt = cnt_tab.reshape(NPAD)[:N_NODES, None]
    wnt = W_neigh.T
    return _tc_combine(x, nb0, nb1, cnt, W_self.T, wnt[:DH], wnt[DH:],
                       bias[None, :], ln_gamma[None, :], ln_beta[None, :])


# speedup vs baseline: 1.3688x; 1.0286x over previous
import functools

import jax
import jax.numpy as jnp
from jax import lax
from jax.experimental import pallas as pl
from jax.experimental.pallas import tpu as pltpu
from jax.experimental.pallas import tpu_sc as plsc

N_NODES = 10000
NPAD = 10240
D = 256
DH = 128
E = 160000
K = 100
ROWS = E // K
NS = 16
TROWS = ROWS // NS
EPT = E // NS
NPT = N_NODES // NS
NBUF = 5
CROWS = NPAD // DH
OFFS = ((0, 100), (100, 100), (200, 100), (300, 100), (400, 100),
        (500, 100), (600, 25))


def _sc_segment_sum(x0, x1, src2, dst2, dst1):
    mesh = plsc.VectorSubcoreMesh(core_axis_name="c", subcore_axis_name="s")

    @functools.partial(
        pl.kernel,
        mesh=mesh,
        compiler_params=pltpu.CompilerParams(use_tc_tiling_on_sc=False,
                                             needs_layout_passes=False),
        out_type=(
            jax.ShapeDtypeStruct((N_NODES, DH), jnp.bfloat16),
            jax.ShapeDtypeStruct((N_NODES, DH), jnp.bfloat16),
            jax.ShapeDtypeStruct((CROWS, DH), jnp.float32),
        ),
        scratch_types=[
            pltpu.VMEM((TROWS, K), jnp.int32),
            pltpu.VMEM((TROWS, K), jnp.int32),
            pltpu.VMEM((EPT,), jnp.int32),
            [pltpu.VMEM((K, DH), jnp.bfloat16) for _ in range(NBUF)],
            pltpu.VMEM((CROWS, DH), jnp.float32),
            pltpu.VMEM((CROWS,), jnp.int32),
            pltpu.VMEM_SHARED((N_NODES, DH), jnp.bfloat16),
            pltpu.VMEM_SHARED((CROWS, DH), jnp.float32),
            [pltpu.SemaphoreType.DMA for _ in range(NBUF)],
            [pltpu.SemaphoreType.DMA for _ in range(NBUF)],
            pltpu.SemaphoreType.DMA,
            pltpu.SemaphoreType.DMA,
        ],
    )
    def k(x0_hbm, x1_hbm, src_hbm, dst_hbm, dst1_hbm, out0, out1, cnt_out,
          sidx, didx, dch, bufs, blk, riota, acc, cnt_sp, semg, sems,
          semi, semw):
        c = lax.axis_index("c")
        s = lax.axis_index("s")

        zero16 = jnp.zeros((16,), jnp.float32)
        one16 = jnp.ones((16,), jnp.float32)
        zero32 = jnp.zeros((32,), jnp.bfloat16)

        pltpu.async_copy(src_hbm.at[pl.ds(s * TROWS, TROWS)], sidx, semi)
        pltpu.async_copy(dst_hbm.at[pl.ds(s * TROWS, TROWS)], didx, semi)
        pltpu.async_copy(dst1_hbm.at[pl.ds(s * EPT, EPT)], dch, semi)

        rows0 = bufs[0]

        def zrow(i, carry):
            for j in range(DH // 32):
                rows0[i, pl.ds(j * 32, 32)] = zero32
            return carry

        lax.fori_loop(0, K, zrow, 0)

        def brow(i, carry):
            for j in range(DH // 16):
                blk[i, pl.ds(j * 16, 16)] = zero16
            return carry

        lax.fori_loop(0, CROWS, brow, 0)

        for j in range(CROWS // 16):
            riota[pl.ds(j * 16, 16)] = lax.iota(jnp.int32, 16) + j * 16

        for o, l in OFFS:
            pltpu.async_copy(rows0.at[pl.ds(0, l)],
                             acc.at[pl.ds(s * NPT + o, l)], semw)

        @pl.when(s == 0)
        def _():
            pltpu.async_copy(blk, cnt_sp, semw)

        for o, l in OFFS:
            pltpu.make_async_copy(rows0.at[pl.ds(0, l)],
                                  acc.at[pl.ds(s * NPT + o, l)], semw).wait()

        @pl.when(s == 0)
        def _():
            pltpu.make_async_copy(blk, cnt_sp, semw).wait()

        pltpu.make_async_copy(src_hbm.at[pl.ds(s * TROWS, TROWS)], sidx,
                              semi).wait()
        pltpu.make_async_copy(dst_hbm.at[pl.ds(s * TROWS, TROWS)], didx,
                              semi).wait()
        pltpu.make_async_copy(dst1_hbm.at[pl.ds(s * EPT, EPT)], dch,
                              semi).wait()

        plsc.subcore_barrier()

        def edge_loop(x_ref):
            for i in range(NBUF):
                pltpu.async_copy(x_ref.at[sidx.at[i]], bufs[i], semg[i])

            def cbody(j, carry):
                idx = dch[pl.ds(j * 16, 16)]
                plsc.addupdate_scatter(
                    blk,
                    [lax.shift_right_logical(idx, 7),
                     lax.bitwise_and(idx, 127)],
                    one16,
                )
                return carry

            lax.fori_loop(0, EPT // 16, cbody, 0)

            def step(q, carry):
                for i in range(NBUF):
                    a = NBUF * q + i
                    j = (i - 1) % NBUF
                    b = a - 1
                    pltpu.make_async_copy(x_ref.at[sidx.at[a]], bufs[i],
                                          semg[i]).wait()

                    @pl.when(b >= 0)
                    def _():
                        pltpu.make_async_copy(bufs[j], acc.at[didx.at[b]],
                                              sems[j]).wait()

                    pltpu.async_copy(bufs[i], acc.at[didx.at[a]], sems[i],
                                     add=True)

                    @pl.when(jnp.logical_and(b >= 0, b + NBUF < TROWS))
                    def _():
                        pltpu.async_copy(x_ref.at[sidx.at[b + NBUF]], bufs[j],
                                         semg[j])
                return carry

            lax.fori_loop(0, TROWS // NBUF, step, 0)

            i = (TROWS - 1) % NBUF
            pltpu.make_async_copy(bufs[i], acc.at[didx.at[TROWS - 1]],
                                  sems[i]).wait()

        @pl.when(c == 0)
        def _():
            edge_loop(x0_hbm)

        @pl.when(c == 1)
        def _():
            edge_loop(x1_hbm)

        plsc.subcore_barrier()

        pltpu.sync_copy(blk, cnt_sp.at[riota], add=True)
        plsc.subcore_barrier()

        def copy_out(dref):
            for n, (o, l) in enumerate(OFFS):
                buf = bufs[n % 2]
                if n >= 2:
                    po, pll = OFFS[n - 2]
                    pltpu.make_async_copy(
                        buf.at[pl.ds(0, pll)],
                        dref.at[pl.ds(s * NPT + po, pll)], semw).wait()
                pltpu.sync_copy(acc.at[pl.ds(s * NPT + o, l)],
                                buf.at[pl.ds(0, l)])
                pltpu.async_copy(buf.at[pl.ds(0, l)],
                                 dref.at[pl.ds(s * NPT + o, l)], semw)
            for n in (len(OFFS) - 2, len(OFFS) - 1):
                o, l = OFFS[n]
                buf = bufs[n % 2]
                pltpu.make_async_copy(buf.at[pl.ds(0, l)],
                                      dref.at[pl.ds(s * NPT + o, l)],
                                      semw).wait()

        @pl.when(c == 0)
        def _():
            copy_out(out0)

        @pl.when(c == 1)
        def _():
            copy_out(out1)

        @pl.when(jnp.logical_and(c == 0, s == 0))
        def _():
            pltpu.sync_copy(cnt_sp, blk)
            pltpu.sync_copy(blk, cnt_out)

    return k(x0, x1, src2, dst2, dst1)


BM = 2000


def _tc_body(x_ref, nb0_ref, nb1_ref, cnt_ref, wst_ref, wnt0_ref, wnt1_ref,
             b_ref, g_ref, be_ref, o_ref):
    hs = jnp.dot(x_ref[...], wst_ref[...], preferred_element_type=jnp.float32)
    hn = (jnp.dot(nb0_ref[...].astype(jnp.float32), wnt0_ref[...],
                  preferred_element_type=jnp.float32)
          + jnp.dot(nb1_ref[...].astype(jnp.float32), wnt1_ref[...],
                    preferred_element_type=jnp.float32))
    inv = 1.0 / jnp.maximum(cnt_ref[...], 1.0)
    h = hs + hn * inv + b_ref[...]
    mu = jnp.mean(h, axis=-1, keepdims=True)
    d = h - mu
    var = jnp.mean(d * d, axis=-1, keepdims=True)
    o_ref[...] = d * lax.rsqrt(var + 1e-5) * g_ref[...] + be_ref[...]


def _tc_combine(x, nb0, nb1, cnt, wst, wnt0, wnt1, bias, gamma, beta):
    grid = (N_NODES // BM,)
    return pl.pallas_call(
        _tc_body,
        grid=grid,
        in_specs=[
            pl.BlockSpec((BM, D), lambda i: (i, 0)),
            pl.BlockSpec((BM, DH), lambda i: (i, 0)),
            pl.BlockSpec((BM, DH), lambda i: (i, 0)),
            pl.BlockSpec((BM, 1), lambda i: (i, 0)),
            pl.BlockSpec((D, D), lambda i: (0, 0)),
            pl.BlockSpec((DH, D), lambda i: (0, 0)),
            pl.BlockSpec((DH, D), lambda i: (0, 0)),
            pl.BlockSpec((1, D), lambda i: (0, 0)),
            pl.BlockSpec((1, D), lambda i: (0, 0)),
            pl.BlockSpec((1, D), lambda i: (0, 0)),
        ],
        out_specs=pl.BlockSpec((BM, D), lambda i: (i, 0)),
        out_shape=jax.ShapeDtypeStruct((N_NODES, D), jnp.float32),
    )(x, nb0, nb1, cnt, wst, wnt0, wnt1, bias, gamma, beta)


@jax.jit
def kernel(x, edge_index, deg, W_self, W_neigh, bias, ln_gamma, ln_beta):
    del deg
    x0 = x[:, :DH].astype(jnp.bfloat16)
    x1 = x[:, DH:].astype(jnp.bfloat16)
    src2 = edge_index[1].reshape(ROWS, K)
    dst2 = edge_index[0].reshape(ROWS, K)
    dst1 = edge_index[0]
    nb0, nb1, cnt_tab = _sc_segment_sum(x0, x1, src2, dst2, dst1)
    cnt = cnt_tab.reshape(NPAD)[:N_NODES, None]
    wnt = W_neigh.T
    return _tc_combine(x, nb0, nb1, cnt, W_self.T, wnt[:DH], wnt[DH:],
                       bias[None, :], ln_gamma[None, :], ln_beta[None, :])
